# Initial kernel scaffold; baseline (speedup 1.0000x reference)
import jax
import jax.numpy as jnp
from jax.experimental import pallas as pl


def kernel(boxes, scores):
    raise NotImplementedError("write your pallas kernel here")


# speedup vs baseline: 13.5641x; 13.5641x over previous
import jax
import jax.numpy as jnp
from jax import lax
from jax.experimental import pallas as pl
from jax.experimental.pallas import tpu as pltpu

_N = 5000
_NPAD = 5120
_THRESH = 0.5


def _nms_body(rows_ref, cols_ref, out_ref):
    cols = cols_ref[:]
    x1c = cols[0:1, :]
    y1c = cols[1:2, :]
    x2c = cols[2:3, :]
    y2c = cols[3:4, :]
    area_c = (x2c - x1c) * (y2c - y1c)
    lidx = lax.broadcasted_iota(jnp.int32, (1, _NPAD), 1)

    def cond(carry):
        cur, _ = carry
        return cur < _N

    def body(carry):
        cur, s = carry
        b = rows_ref[pl.ds(cur, 1), :]
        x1 = b[:, 0:1]
        y1 = b[:, 1:2]
        x2 = b[:, 2:3]
        y2 = b[:, 3:4]
        area_r = (x2 - x1) * (y2 - y1)
        ltx = jnp.maximum(x1, x1c)
        lty = jnp.maximum(y1, y1c)
        rbx = jnp.minimum(x2, x2c)
        rby = jnp.minimum(y2, y2c)
        w = jnp.maximum(rbx - ltx, 0.0)
        h = jnp.maximum(rby - lty, 0.0)
        inter = w * h
        union = area_r + area_c - inter
        iou = inter / jnp.maximum(union, 1e-9)
        later = lidx > cur
        supp = jnp.where((iou > _THRESH) & later, 1.0, 0.0)
        s = jnp.maximum(s, supp)
        cand = jnp.where((s == 0.0) & later, lidx, jnp.int32(_NPAD))
        nxt = jnp.min(cand)
        return nxt, s

    s0 = jnp.zeros((1, _NPAD), dtype=jnp.float32)
    _, s = lax.while_loop(cond, body, (jnp.int32(0), s0))
    keep = s == 0.0
    out_ref[:, :] = jnp.where(keep, cols, 0.0)


def kernel(boxes, scores):
    order = jnp.argsort(-scores)
    sb = jnp.take(boxes, order, axis=0)
    ss = jnp.take(scores, order, axis=0)
    pad = _NPAD - _N
    sbp = jnp.pad(sb, ((0, pad), (0, 0)))
    ssp = jnp.pad(ss, (0, pad))
    cc = jnp.concatenate([sbp.T, ssp[None, :]], axis=0)
    out = pl.pallas_call(
        _nms_body,
        out_shape=jax.ShapeDtypeStruct((5, _NPAD), jnp.float32),
        in_specs=[
            pl.BlockSpec(memory_space=pltpu.VMEM),
            pl.BlockSpec(memory_space=pltpu.VMEM),
        ],
        out_specs=pl.BlockSpec(memory_space=pltpu.VMEM),
    )(sbp, cc)
    return out.T[:_N]


# speedup vs baseline: 20.4645x; 1.5087x over previous
import jax
import jax.numpy as jnp
from jax import lax
from jax.experimental import pallas as pl
from jax.experimental.pallas import tpu as pltpu

_N = 5000
_NPAD = 5120
_R = 8
_C = _NPAD // _R
_THRESH = 0.5


def _nms_body(rows_ref, cols_ref, out_ref):
    cols = cols_ref[:]
    x1c = cols[0]
    y1c = cols[1]
    x2c = cols[2]
    y2c = cols[3]
    area_c = (x2c - x1c) * (y2c - y1c)
    lidx = (lax.broadcasted_iota(jnp.int32, (_R, _C), 0) * _C
            + lax.broadcasted_iota(jnp.int32, (_R, _C), 1))

    def cond(carry):
        cur, _ = carry
        return cur < _N

    def body(carry):
        cur, s = carry
        b = rows_ref[pl.ds(cur, 1), :]
        x1 = b[0, 0]
        y1 = b[0, 1]
        x2 = b[0, 2]
        y2 = b[0, 3]
        area_r = (x2 - x1) * (y2 - y1)
        ltx = jnp.maximum(x1, x1c)
        lty = jnp.maximum(y1, y1c)
        rbx = jnp.minimum(x2, x2c)
        rby = jnp.minimum(y2, y2c)
        w = jnp.maximum(rbx - ltx, 0.0)
        h = jnp.maximum(rby - lty, 0.0)
        inter = w * h
        union = area_r + area_c - inter
        iou = inter / jnp.maximum(union, 1e-9)
        later = lidx > cur
        hit = (iou > _THRESH) & later
        cand = jnp.where(later & (s == 0.0) & (~hit), lidx, jnp.int32(_NPAD))
        nxt = jnp.min(cand)
        s = jnp.maximum(s, jnp.where(hit, 1.0, 0.0))
        return nxt, s

    s0 = jnp.zeros((_R, _C), dtype=jnp.float32)
    _, s = lax.while_loop(cond, body, (jnp.int32(0), s0))
    keep = s == 0.0
    out_ref[:] = jnp.where(keep[None], cols, 0.0)


def kernel(boxes, scores):
    order = jnp.argsort(-scores)
    sb = jnp.take(boxes, order, axis=0)
    ss = jnp.take(scores, order, axis=0)
    pad = _NPAD - _N
    sbp = jnp.pad(sb, ((0, pad), (0, 0)))
    ssp = jnp.pad(ss, (0, pad))
    cc = jnp.concatenate([sbp.T, ssp[None, :]], axis=0)
    cc3 = cc.reshape(5, _R, _C)
    out = pl.pallas_call(
        _nms_body,
        out_shape=jax.ShapeDtypeStruct((5, _R, _C), jnp.float32),
        in_specs=[
            pl.BlockSpec(memory_space=pltpu.VMEM),
            pl.BlockSpec(memory_space=pltpu.VMEM),
        ],
        out_specs=pl.BlockSpec(memory_space=pltpu.VMEM),
    )(sbp, cc3)
    return out.reshape(5, _NPAD).T[:_N]


# speedup vs baseline: 41.9850x; 2.0516x over previous
import jax
import jax.numpy as jnp
from jax import lax
from jax.experimental import pallas as pl
from jax.experimental.pallas import tpu as pltpu

_N = 5000
_NPAD = 5120
_R = 40
_C = 128
_THRESH = 0.5


def _nms_body(splat_ref, cols_ref, out_ref):
    cols = cols_ref[:]
    x1c = cols[0]
    y1c = cols[1]
    x2c = cols[2]
    y2c = cols[3]
    area_c = (x2c - x1c) * (y2c - y1c)
    lidx = (lax.broadcasted_iota(jnp.int32, (_R, _C), 0) * _C
            + lax.broadcasted_iota(jnp.int32, (_R, _C), 1)
            ).astype(jnp.float32)

    def cond(carry):
        cur, _ = carry
        return cur < _N

    def body(carry):
        cur, s = carry
        curf = cur.astype(jnp.float32)
        x1 = splat_ref[0, pl.ds(cur, 1), :]
        y1 = splat_ref[1, pl.ds(cur, 1), :]
        x2 = splat_ref[2, pl.ds(cur, 1), :]
        y2 = splat_ref[3, pl.ds(cur, 1), :]
        area_r = (x2 - x1) * (y2 - y1)
        ltx = jnp.maximum(x1, x1c)
        lty = jnp.maximum(y1, y1c)
        rbx = jnp.minimum(x2, x2c)
        rby = jnp.minimum(y2, y2c)
        w = jnp.maximum(rbx - ltx, 0.0)
        h = jnp.maximum(rby - lty, 0.0)
        inter = w * h
        union = area_r + area_c - inter
        later = lidx > curf
        hit = (inter > _THRESH * union) & later
        cand = jnp.where(later & (s == 0.0) & (~hit), lidx,
                         jnp.float32(_NPAD))
        m = jnp.min(cand, axis=0)
        nxt = jnp.min(m).astype(jnp.int32)
        s = jnp.maximum(s, jnp.where(hit, 1.0, 0.0))
        return nxt, s

    s0 = jnp.zeros((_R, _C), dtype=jnp.float32)
    _, s = lax.while_loop(cond, body, (jnp.int32(0), s0))
    keep = s == 0.0
    out_ref[:] = jnp.where(keep[None], cols, 0.0)


def kernel(boxes, scores):
    order = jnp.argsort(-scores)
    sb = jnp.take(boxes, order, axis=0)
    ss = jnp.take(scores, order, axis=0)
    pad = _NPAD - _N
    sbp = jnp.pad(sb, ((0, pad), (0, 0)))
    ssp = jnp.pad(ss, (0, pad))
    cc = jnp.concatenate([sbp.T, ssp[None, :]], axis=0)
    cc3 = cc.reshape(5, _R, _C)
    splat = jnp.broadcast_to(sbp.T[:, :, None], (4, _NPAD, _C))
    out = pl.pallas_call(
        _nms_body,
        out_shape=jax.ShapeDtypeStruct((5, _R, _C), jnp.float32),
        in_specs=[
            pl.BlockSpec(memory_space=pltpu.VMEM),
            pl.BlockSpec(memory_space=pltpu.VMEM),
        ],
        out_specs=pl.BlockSpec(memory_space=pltpu.VMEM),
    )(splat, cc3)
    return out.reshape(5, _NPAD).T[:_N]


# speedup vs baseline: 43.5023x; 1.0361x over previous
import jax
import jax.numpy as jnp
from jax import lax
from jax.experimental import pallas as pl
from jax.experimental.pallas import tpu as pltpu

_N = 5000
_NPAD = 5120
_R = 40
_C = 128
_THRESH = 0.5


def _nms_body(splat_ref, cols_ref, out_ref):
    cols = cols_ref[:]
    x1c = cols[0]
    y1c = cols[1]
    x2c = cols[2]
    y2c = cols[3]
    area_c = (x2c - x1c) * (y2c - y1c)
    lidx = (lax.broadcasted_iota(jnp.int32, (_R, _C), 0) * _C
            + lax.broadcasted_iota(jnp.int32, (_R, _C), 1)
            ).astype(jnp.float32)

    def cond(carry):
        cur, _ = carry
        return cur < _N

    def body(carry):
        cur, c = carry
        curf = cur.astype(jnp.float32)
        x1 = splat_ref[0, pl.ds(cur, 1), :]
        y1 = splat_ref[1, pl.ds(cur, 1), :]
        x2 = splat_ref[2, pl.ds(cur, 1), :]
        y2 = splat_ref[3, pl.ds(cur, 1), :]
        area_r = (x2 - x1) * (y2 - y1)
        ltx = jnp.maximum(x1, x1c)
        lty = jnp.maximum(y1, y1c)
        rbx = jnp.minimum(x2, x2c)
        rby = jnp.minimum(y2, y2c)
        w = jnp.maximum(rbx - ltx, 0.0)
        h = jnp.maximum(rby - lty, 0.0)
        inter = w * h
        union = area_r + area_c - inter
        later = lidx > curf
        hit = (inter > _THRESH * union) & later
        c = jnp.where(hit, jnp.float32(_NPAD), c)
        cand = jnp.where(later, c, jnp.float32(_NPAD))
        m = jnp.min(cand, axis=0)
        nxt = jnp.min(m).astype(jnp.int32)
        return nxt, c

    _, c = lax.while_loop(cond, body, (jnp.int32(0), lidx))
    keep = c == lidx
    out_ref[:] = jnp.where(keep[None], cols, 0.0)


def kernel(boxes, scores):
    order = jnp.argsort(-scores)
    sb = jnp.take(boxes, order, axis=0)
    ss = jnp.take(scores, order, axis=0)
    pad = _NPAD - _N
    sbp = jnp.pad(sb, ((0, pad), (0, 0)))
    ssp = jnp.pad(ss, (0, pad))
    cc = jnp.concatenate([sbp.T, ssp[None, :]], axis=0)
    cc3 = cc.reshape(5, _R, _C)
    splat = jnp.broadcast_to(sbp.T[:, :, None], (4, _NPAD, _C))
    out = pl.pallas_call(
        _nms_body,
        out_shape=jax.ShapeDtypeStruct((5, _R, _C), jnp.float32),
        in_specs=[
            pl.BlockSpec(memory_space=pltpu.VMEM),
            pl.BlockSpec(memory_space=pltpu.VMEM),
        ],
        out_specs=pl.BlockSpec(memory_space=pltpu.VMEM),
    )(splat, cc3)
    return out.reshape(5, _NPAD).T[:_N]
